# initial kernel scaffold (unmeasured)
import jax
import jax.numpy as jnp
from jax import lax
from jax.experimental import pallas as pl
from jax.experimental.pallas import tpu as pltpu


def kernel(
    x,
):
    def body(*refs):
        pass

    out_shape = jax.ShapeDtypeStruct(..., jnp.float32)
    return pl.pallas_call(body, out_shape=out_shape)(...)


# baseline (device time: 54649 ns/iter reference)
import jax
import jax.numpy as jnp
from jax import lax
from jax.experimental import pallas as pl
from jax.experimental.pallas import tpu as pltpu


def kernel(x):
    _, m, n_total = x.shape
    n_out = n_total // 2

    def body(x_ref, out_ref, comm_ref, send_sem, recv_sem):
        my_x = lax.axis_index("x")
        my_y = lax.axis_index("y")
        my_z = lax.axis_index("z")
        peer = (my_x, 1 - my_y, my_z)

        barrier_sem = pltpu.get_barrier_semaphore()
        pl.semaphore_signal(
            barrier_sem, inc=1,
            device_id=peer, device_id_type=pl.DeviceIdType.MESH,
        )
        pl.semaphore_wait(barrier_sem, 1)

        rdma = pltpu.make_async_remote_copy(
            src_ref=x_ref.at[0, :, pl.ds((1 - my_y) * n_out, n_out)],
            dst_ref=comm_ref,
            send_sem=send_sem,
            recv_sem=recv_sem,
            device_id=peer,
            device_id_type=pl.DeviceIdType.MESH,
        )
        rdma.start()
        rdma.wait()

        out_ref[:, :] = (
            x_ref[0, :, pl.ds(my_y * n_out, n_out)] + comm_ref[:, :]
        )

    return pl.pallas_call(
        body,
        out_shape=jax.ShapeDtypeStruct((m, n_out), x.dtype),
        in_specs=[pl.BlockSpec(memory_space=pltpu.VMEM)],
        out_specs=pl.BlockSpec(memory_space=pltpu.VMEM),
        scratch_shapes=[
            pltpu.VMEM((m, n_out), x.dtype),
            pltpu.SemaphoreType.DMA,
            pltpu.SemaphoreType.DMA,
        ],
        compiler_params=pltpu.CompilerParams(collective_id=0),
    )(x)


# device time: 37304 ns/iter; 1.4650x vs baseline; 1.4650x over previous
import jax
import jax.numpy as jnp
from jax import lax
from jax.experimental import pallas as pl
from jax.experimental.pallas import tpu as pltpu

N_CHUNKS = 8


def kernel(x):
    _, m, n_total = x.shape
    n_out = n_total // 2
    half = m // 2
    chunk = half // N_CHUNKS

    def body(x_ref, out_ref, ybuf, zbuf, ysend, yrecv, zsend, zrecv):
        my_x = lax.axis_index("x")
        my_y = lax.axis_index("y")
        my_z = lax.axis_index("z")
        peer = (my_x, 1 - my_y, my_z)
        znb = (my_x, my_y, 1 - my_z)

        barrier_sem = pltpu.get_barrier_semaphore()
        for nbr in (peer, znb):
            pl.semaphore_signal(
                barrier_sem, inc=1,
                device_id=nbr, device_id_type=pl.DeviceIdType.MESH,
            )
        pl.semaphore_wait(barrier_sem, 2)

        my_half = my_z * half
        other_half = (1 - my_z) * half
        my_cols = pl.ds(my_y * n_out, n_out)
        peer_cols = pl.ds((1 - my_y) * n_out, n_out)

        y_rdmas = []
        for k in range(N_CHUNKS):
            r = pltpu.make_async_remote_copy(
                src_ref=x_ref.at[0, pl.ds(my_half + k * chunk, chunk), peer_cols],
                dst_ref=ybuf.at[pl.ds(k * chunk, chunk), :],
                send_sem=ysend.at[k],
                recv_sem=yrecv.at[k],
                device_id=peer,
                device_id_type=pl.DeviceIdType.MESH,
            )
            r.start()
            y_rdmas.append(r)

        z_rdmas = []
        for k in range(N_CHUNKS):
            y_rdmas[k].wait_recv()
            rz = pltpu.make_async_remote_copy(
                src_ref=ybuf.at[pl.ds(k * chunk, chunk), :],
                dst_ref=zbuf.at[pl.ds(k * chunk, chunk), :],
                send_sem=zsend.at[k],
                recv_sem=zrecv.at[k],
                device_id=znb,
                device_id_type=pl.DeviceIdType.MESH,
            )
            rz.start()
            z_rdmas.append(rz)
            rows = pl.ds(my_half + k * chunk, chunk)
            out_ref[rows, :] = (
                x_ref[0, rows, my_cols] + ybuf[pl.ds(k * chunk, chunk), :]
            )

        for k in range(N_CHUNKS):
            z_rdmas[k].wait_recv()
            rows = pl.ds(other_half + k * chunk, chunk)
            out_ref[rows, :] = (
                x_ref[0, rows, my_cols] + zbuf[pl.ds(k * chunk, chunk), :]
            )

        for k in range(N_CHUNKS):
            y_rdmas[k].wait_send()
            z_rdmas[k].wait_send()

    return pl.pallas_call(
        body,
        out_shape=jax.ShapeDtypeStruct((m, n_out), x.dtype),
        in_specs=[pl.BlockSpec(memory_space=pltpu.VMEM)],
        out_specs=pl.BlockSpec(memory_space=pltpu.VMEM),
        scratch_shapes=[
            pltpu.VMEM((half, n_out), x.dtype),
            pltpu.VMEM((half, n_out), x.dtype),
            pltpu.SemaphoreType.DMA((N_CHUNKS,)),
            pltpu.SemaphoreType.DMA((N_CHUNKS,)),
            pltpu.SemaphoreType.DMA((N_CHUNKS,)),
            pltpu.SemaphoreType.DMA((N_CHUNKS,)),
        ],
        compiler_params=pltpu.CompilerParams(collective_id=0),
    )(x)


# device time: 32717 ns/iter; 1.6704x vs baseline; 1.1402x over previous
import jax
import jax.numpy as jnp
from jax import lax
from jax.experimental import pallas as pl
from jax.experimental.pallas import tpu as pltpu

N_CHUNKS = 4


def kernel(x):
    _, m, n_total = x.shape
    n_out = n_total // 2
    quarter = m // 4
    chunk = quarter // N_CHUNKS
    n_relay = N_CHUNKS // 2

    def body(x_ref, out_ref, ybuf, xdbuf, zdbuf, xrbuf, zrbuf,
             ysend, yrecv, xdsend, xdrecv, zdsend, zdrecv,
             xrsend, xrrecv, zrsend, zrrecv):
        my_x = lax.axis_index("x")
        my_y = lax.axis_index("y")
        my_z = lax.axis_index("z")
        peer = (my_x, 1 - my_y, my_z)
        xnb = (1 - my_x, my_y, my_z)
        znb = (my_x, my_y, 1 - my_z)

        barrier_sem = pltpu.get_barrier_semaphore()
        for nbr in (peer, xnb, znb):
            pl.semaphore_signal(
                barrier_sem, inc=1,
                device_id=nbr, device_id_type=pl.DeviceIdType.MESH,
            )
        pl.semaphore_wait(barrier_sem, 3)

        my_cols = pl.ds(my_y * n_out, n_out)
        peer_cols = pl.ds((1 - my_y) * n_out, n_out)
        q_mine = (2 * my_x + my_z) * quarter
        q_xnb = (2 * (1 - my_x) + my_z) * quarter
        q_znb = (2 * my_x + (1 - my_z)) * quarter
        q_diag = (2 * (1 - my_x) + (1 - my_z)) * quarter

        def cs(base, c):
            return pl.ds(base + c * chunk, chunk)

        y_rdmas = []
        for c in range(N_CHUNKS):
            r = pltpu.make_async_remote_copy(
                src_ref=x_ref.at[0, cs(q_mine, c), peer_cols],
                dst_ref=ybuf.at[cs(0, c), :],
                send_sem=ysend.at[c],
                recv_sem=yrecv.at[c],
                device_id=peer,
                device_id_type=pl.DeviceIdType.MESH,
            )
            r.start()
            y_rdmas.append(r)

        xd_rdmas, zd_rdmas = [], []
        for c in range(N_CHUNKS):
            y_rdmas[c].wait_recv()
            rx = pltpu.make_async_remote_copy(
                src_ref=ybuf.at[cs(0, c), :],
                dst_ref=xdbuf.at[cs(0, c), :],
                send_sem=xdsend.at[c],
                recv_sem=xdrecv.at[c],
                device_id=xnb,
                device_id_type=pl.DeviceIdType.MESH,
            )
            rx.start()
            xd_rdmas.append(rx)
            rz = pltpu.make_async_remote_copy(
                src_ref=ybuf.at[cs(0, c), :],
                dst_ref=zdbuf.at[cs(0, c), :],
                send_sem=zdsend.at[c],
                recv_sem=zdrecv.at[c],
                device_id=znb,
                device_id_type=pl.DeviceIdType.MESH,
            )
            rz.start()
            zd_rdmas.append(rz)
            out_ref[cs(q_mine, c), :] = (
                x_ref[0, cs(q_mine, c), my_cols] + ybuf[cs(0, c), :]
            )

        xr_rdmas, zr_rdmas = [], []
        for c in range(N_CHUNKS):
            zd_rdmas[c].wait_recv()
            if c % 2 == 0:
                j = c // 2
                rr = pltpu.make_async_remote_copy(
                    src_ref=zdbuf.at[cs(0, c), :],
                    dst_ref=xrbuf.at[cs(0, j), :],
                    send_sem=xrsend.at[j],
                    recv_sem=xrrecv.at[j],
                    device_id=xnb,
                    device_id_type=pl.DeviceIdType.MESH,
                )
                rr.start()
                xr_rdmas.append(rr)
            out_ref[cs(q_znb, c), :] = (
                x_ref[0, cs(q_znb, c), my_cols] + zdbuf[cs(0, c), :]
            )
            xd_rdmas[c].wait_recv()
            if c % 2 == 1:
                j = (c - 1) // 2
                rr = pltpu.make_async_remote_copy(
                    src_ref=xdbuf.at[cs(0, c), :],
                    dst_ref=zrbuf.at[cs(0, j), :],
                    send_sem=zrsend.at[j],
                    recv_sem=zrrecv.at[j],
                    device_id=znb,
                    device_id_type=pl.DeviceIdType.MESH,
                )
                rr.start()
                zr_rdmas.append(rr)
            out_ref[cs(q_xnb, c), :] = (
                x_ref[0, cs(q_xnb, c), my_cols] + xdbuf[cs(0, c), :]
            )

        for j in range(n_relay):
            xr_rdmas[j].wait_recv()
            out_ref[cs(q_diag, 2 * j), :] = (
                x_ref[0, cs(q_diag, 2 * j), my_cols] + xrbuf[cs(0, j), :]
            )
            zr_rdmas[j].wait_recv()
            out_ref[cs(q_diag, 2 * j + 1), :] = (
                x_ref[0, cs(q_diag, 2 * j + 1), my_cols] + zrbuf[cs(0, j), :]
            )

        for c in range(N_CHUNKS):
            y_rdmas[c].wait_send()
            xd_rdmas[c].wait_send()
            zd_rdmas[c].wait_send()
        for j in range(n_relay):
            xr_rdmas[j].wait_send()
            zr_rdmas[j].wait_send()

    return pl.pallas_call(
        body,
        out_shape=jax.ShapeDtypeStruct((m, n_out), x.dtype),
        in_specs=[pl.BlockSpec(memory_space=pltpu.VMEM)],
        out_specs=pl.BlockSpec(memory_space=pltpu.VMEM),
        scratch_shapes=[
            pltpu.VMEM((quarter, n_out), x.dtype),
            pltpu.VMEM((quarter, n_out), x.dtype),
            pltpu.VMEM((quarter, n_out), x.dtype),
            pltpu.VMEM((quarter // 2, n_out), x.dtype),
            pltpu.VMEM((quarter // 2, n_out), x.dtype),
            pltpu.SemaphoreType.DMA((N_CHUNKS,)),
            pltpu.SemaphoreType.DMA((N_CHUNKS,)),
            pltpu.SemaphoreType.DMA((N_CHUNKS,)),
            pltpu.SemaphoreType.DMA((N_CHUNKS,)),
            pltpu.SemaphoreType.DMA((N_CHUNKS,)),
            pltpu.SemaphoreType.DMA((N_CHUNKS,)),
            pltpu.SemaphoreType.DMA((N_CHUNKS // 2,)),
            pltpu.SemaphoreType.DMA((N_CHUNKS // 2,)),
            pltpu.SemaphoreType.DMA((N_CHUNKS // 2,)),
            pltpu.SemaphoreType.DMA((N_CHUNKS // 2,)),
        ],
        compiler_params=pltpu.CompilerParams(collective_id=0),
    )(x)


# device time: 31631 ns/iter; 1.7277x vs baseline; 1.0343x over previous
import jax
import jax.numpy as jnp
from jax import lax
from jax.experimental import pallas as pl
from jax.experimental.pallas import tpu as pltpu

N_CHUNKS = 8


def kernel(x):
    _, m, n_total = x.shape
    n_out = n_total // 2
    quarter = m // 4
    chunk = quarter // N_CHUNKS
    n_relay = N_CHUNKS // 2

    def body(x_ref, out_ref, ybuf, xdbuf, zdbuf, xrbuf, zrbuf,
             ysend, yrecv, xdsend, xdrecv, zdsend, zdrecv,
             xrsend, xrrecv, zrsend, zrrecv):
        my_x = lax.axis_index("x")
        my_y = lax.axis_index("y")
        my_z = lax.axis_index("z")
        peer = (my_x, 1 - my_y, my_z)
        xnb = (1 - my_x, my_y, my_z)
        znb = (my_x, my_y, 1 - my_z)

        barrier_sem = pltpu.get_barrier_semaphore()
        for nbr in (peer, xnb, znb):
            pl.semaphore_signal(
                barrier_sem, inc=1,
                device_id=nbr, device_id_type=pl.DeviceIdType.MESH,
            )
        pl.semaphore_wait(barrier_sem, 3)

        my_cols = pl.ds(my_y * n_out, n_out)
        peer_cols = pl.ds((1 - my_y) * n_out, n_out)
        q_mine = (2 * my_x + my_z) * quarter
        q_xnb = (2 * (1 - my_x) + my_z) * quarter
        q_znb = (2 * my_x + (1 - my_z)) * quarter
        q_diag = (2 * (1 - my_x) + (1 - my_z)) * quarter

        def cs(base, c):
            return pl.ds(base + c * chunk, chunk)

        y_rdmas = []
        for c in range(N_CHUNKS):
            r = pltpu.make_async_remote_copy(
                src_ref=x_ref.at[0, cs(q_mine, c), peer_cols],
                dst_ref=ybuf.at[cs(0, c), :],
                send_sem=ysend.at[c],
                recv_sem=yrecv.at[c],
                device_id=peer,
                device_id_type=pl.DeviceIdType.MESH,
            )
            r.start()
            y_rdmas.append(r)

        xd_rdmas, zd_rdmas = [], []
        for c in range(N_CHUNKS):
            y_rdmas[c].wait_recv()
            rx = pltpu.make_async_remote_copy(
                src_ref=ybuf.at[cs(0, c), :],
                dst_ref=xdbuf.at[cs(0, c), :],
                send_sem=xdsend.at[c],
                recv_sem=xdrecv.at[c],
                device_id=xnb,
                device_id_type=pl.DeviceIdType.MESH,
            )
            rx.start()
            xd_rdmas.append(rx)
            rz = pltpu.make_async_remote_copy(
                src_ref=ybuf.at[cs(0, c), :],
                dst_ref=zdbuf.at[cs(0, c), :],
                send_sem=zdsend.at[c],
                recv_sem=zdrecv.at[c],
                device_id=znb,
                device_id_type=pl.DeviceIdType.MESH,
            )
            rz.start()
            zd_rdmas.append(rz)
            out_ref[cs(q_mine, c), :] = (
                x_ref[0, cs(q_mine, c), my_cols] + ybuf[cs(0, c), :]
            )

        xr_rdmas, zr_rdmas = [], []
        for c in range(N_CHUNKS):
            zd_rdmas[c].wait_recv()
            if c % 2 == 0:
                j = c // 2
                rr = pltpu.make_async_remote_copy(
                    src_ref=zdbuf.at[cs(0, c), :],
                    dst_ref=xrbuf.at[cs(0, j), :],
                    send_sem=xrsend.at[j],
                    recv_sem=xrrecv.at[j],
                    device_id=xnb,
                    device_id_type=pl.DeviceIdType.MESH,
                )
                rr.start()
                xr_rdmas.append(rr)
            out_ref[cs(q_znb, c), :] = (
                x_ref[0, cs(q_znb, c), my_cols] + zdbuf[cs(0, c), :]
            )
            xd_rdmas[c].wait_recv()
            if c % 2 == 1:
                j = (c - 1) // 2
                rr = pltpu.make_async_remote_copy(
                    src_ref=xdbuf.at[cs(0, c), :],
                    dst_ref=zrbuf.at[cs(0, j), :],
                    send_sem=zrsend.at[j],
                    recv_sem=zrrecv.at[j],
                    device_id=znb,
                    device_id_type=pl.DeviceIdType.MESH,
                )
                rr.start()
                zr_rdmas.append(rr)
            out_ref[cs(q_xnb, c), :] = (
                x_ref[0, cs(q_xnb, c), my_cols] + xdbuf[cs(0, c), :]
            )

        for j in range(n_relay):
            xr_rdmas[j].wait_recv()
            out_ref[cs(q_diag, 2 * j), :] = (
                x_ref[0, cs(q_diag, 2 * j), my_cols] + xrbuf[cs(0, j), :]
            )
            zr_rdmas[j].wait_recv()
            out_ref[cs(q_diag, 2 * j + 1), :] = (
                x_ref[0, cs(q_diag, 2 * j + 1), my_cols] + zrbuf[cs(0, j), :]
            )

        for c in range(N_CHUNKS):
            y_rdmas[c].wait_send()
            xd_rdmas[c].wait_send()
            zd_rdmas[c].wait_send()
        for j in range(n_relay):
            xr_rdmas[j].wait_send()
            zr_rdmas[j].wait_send()

    return pl.pallas_call(
        body,
        out_shape=jax.ShapeDtypeStruct((m, n_out), x.dtype),
        in_specs=[pl.BlockSpec(memory_space=pltpu.VMEM)],
        out_specs=pl.BlockSpec(memory_space=pltpu.VMEM),
        scratch_shapes=[
            pltpu.VMEM((quarter, n_out), x.dtype),
            pltpu.VMEM((quarter, n_out), x.dtype),
            pltpu.VMEM((quarter, n_out), x.dtype),
            pltpu.VMEM((quarter // 2, n_out), x.dtype),
            pltpu.VMEM((quarter // 2, n_out), x.dtype),
            pltpu.SemaphoreType.DMA((N_CHUNKS,)),
            pltpu.SemaphoreType.DMA((N_CHUNKS,)),
            pltpu.SemaphoreType.DMA((N_CHUNKS,)),
            pltpu.SemaphoreType.DMA((N_CHUNKS,)),
            pltpu.SemaphoreType.DMA((N_CHUNKS,)),
            pltpu.SemaphoreType.DMA((N_CHUNKS,)),
            pltpu.SemaphoreType.DMA((N_CHUNKS // 2,)),
            pltpu.SemaphoreType.DMA((N_CHUNKS // 2,)),
            pltpu.SemaphoreType.DMA((N_CHUNKS // 2,)),
            pltpu.SemaphoreType.DMA((N_CHUNKS // 2,)),
        ],
        compiler_params=pltpu.CompilerParams(collective_id=0),
    )(x)


# device time: 30224 ns/iter; 1.8081x vs baseline; 1.0466x over previous
import jax
import jax.numpy as jnp
from jax import lax
from jax.experimental import pallas as pl
from jax.experimental.pallas import tpu as pltpu

N_CHUNKS = 8
VIA_X = (0, 2, 4)
VIA_Z = (1, 3, 5)
VIA_Y = (6, 7)


def kernel(x):
    _, m, n_total = x.shape
    n_out = n_total // 2
    quarter = m // 4
    chunk = quarter // N_CHUNKS

    def body(x_ref, out_ref, ybuf, ydbuf, xdbuf, zdbuf, xrbuf, zrbuf,
             ysend, yrecv, ydsend, ydrecv, xdsend, xdrecv, zdsend, zdrecv,
             xrsend, xrrecv, zrsend, zrrecv):
        my_x = lax.axis_index("x")
        my_y = lax.axis_index("y")
        my_z = lax.axis_index("z")
        peer = (my_x, 1 - my_y, my_z)
        xnb = (1 - my_x, my_y, my_z)
        znb = (my_x, my_y, 1 - my_z)

        barrier_sem = pltpu.get_barrier_semaphore()
        for nbr in (peer, xnb, znb):
            pl.semaphore_signal(
                barrier_sem, inc=1,
                device_id=nbr, device_id_type=pl.DeviceIdType.MESH,
            )
        pl.semaphore_wait(barrier_sem, 3)

        my_cols = pl.ds(my_y * n_out, n_out)
        peer_cols = pl.ds((1 - my_y) * n_out, n_out)
        q_mine = (2 * my_x + my_z) * quarter
        q_xnb = (2 * (1 - my_x) + my_z) * quarter
        q_znb = (2 * my_x + (1 - my_z)) * quarter
        q_diag = (2 * (1 - my_x) + (1 - my_z)) * quarter

        def cs(base, c):
            return pl.ds(base + c * chunk, chunk)

        y_rdmas = []
        for c in range(N_CHUNKS):
            r = pltpu.make_async_remote_copy(
                src_ref=x_ref.at[0, cs(q_mine, c), peer_cols],
                dst_ref=ybuf.at[cs(0, c), :],
                send_sem=ysend.at[c],
                recv_sem=yrecv.at[c],
                device_id=peer,
                device_id_type=pl.DeviceIdType.MESH,
            )
            r.start()
            y_rdmas.append(r)
        yd_rdmas = []
        for j, c in enumerate(VIA_Y):
            r = pltpu.make_async_remote_copy(
                src_ref=x_ref.at[0, cs(q_diag, c), peer_cols],
                dst_ref=ydbuf.at[cs(0, j), :],
                send_sem=ydsend.at[j],
                recv_sem=ydrecv.at[j],
                device_id=peer,
                device_id_type=pl.DeviceIdType.MESH,
            )
            r.start()
            yd_rdmas.append(r)

        xd_rdmas, zd_rdmas = [], []
        for c in range(N_CHUNKS):
            y_rdmas[c].wait_recv()
            rx = pltpu.make_async_remote_copy(
                src_ref=ybuf.at[cs(0, c), :],
                dst_ref=xdbuf.at[cs(0, c), :],
                send_sem=xdsend.at[c],
                recv_sem=xdrecv.at[c],
                device_id=xnb,
                device_id_type=pl.DeviceIdType.MESH,
            )
            rx.start()
            xd_rdmas.append(rx)
            rz = pltpu.make_async_remote_copy(
                src_ref=ybuf.at[cs(0, c), :],
                dst_ref=zdbuf.at[cs(0, c), :],
                send_sem=zdsend.at[c],
                recv_sem=zdrecv.at[c],
                device_id=znb,
                device_id_type=pl.DeviceIdType.MESH,
            )
            rz.start()
            zd_rdmas.append(rz)
            out_ref[cs(q_mine, c), :] = (
                x_ref[0, cs(q_mine, c), my_cols] + ybuf[cs(0, c), :]
            )

        xr_rdmas, zr_rdmas = [], []
        for c in range(N_CHUNKS):
            zd_rdmas[c].wait_recv()
            if c in VIA_X:
                j = VIA_X.index(c)
                rr = pltpu.make_async_remote_copy(
                    src_ref=zdbuf.at[cs(0, c), :],
                    dst_ref=xrbuf.at[cs(0, j), :],
                    send_sem=xrsend.at[j],
                    recv_sem=xrrecv.at[j],
                    device_id=xnb,
                    device_id_type=pl.DeviceIdType.MESH,
                )
                rr.start()
                xr_rdmas.append(rr)
            out_ref[cs(q_znb, c), :] = (
                x_ref[0, cs(q_znb, c), my_cols] + zdbuf[cs(0, c), :]
            )
            xd_rdmas[c].wait_recv()
            if c in VIA_Z:
                j = VIA_Z.index(c)
                rr = pltpu.make_async_remote_copy(
                    src_ref=xdbuf.at[cs(0, c), :],
                    dst_ref=zrbuf.at[cs(0, j), :],
                    send_sem=zrsend.at[j],
                    recv_sem=zrrecv.at[j],
                    device_id=znb,
                    device_id_type=pl.DeviceIdType.MESH,
                )
                rr.start()
                zr_rdmas.append(rr)
            out_ref[cs(q_xnb, c), :] = (
                x_ref[0, cs(q_xnb, c), my_cols] + xdbuf[cs(0, c), :]
            )

        for j in range(len(VIA_X)):
            xr_rdmas[j].wait_recv()
            c = VIA_X[j]
            out_ref[cs(q_diag, c), :] = (
                x_ref[0, cs(q_diag, c), my_cols] + xrbuf[cs(0, j), :]
            )
            zr_rdmas[j].wait_recv()
            c = VIA_Z[j]
            out_ref[cs(q_diag, c), :] = (
                x_ref[0, cs(q_diag, c), my_cols] + zrbuf[cs(0, j), :]
            )
        for j in range(len(VIA_Y)):
            yd_rdmas[j].wait_recv()
            c = VIA_Y[j]
            out_ref[cs(q_diag, c), :] = (
                x_ref[0, cs(q_diag, c), my_cols] + ydbuf[cs(0, j), :]
            )

        for c in range(N_CHUNKS):
            y_rdmas[c].wait_send()
            xd_rdmas[c].wait_send()
            zd_rdmas[c].wait_send()
        for j in range(len(VIA_X)):
            xr_rdmas[j].wait_send()
            zr_rdmas[j].wait_send()
        for j in range(len(VIA_Y)):
            yd_rdmas[j].wait_send()

    n_x = len(VIA_X)
    n_z = len(VIA_Z)
    n_y = len(VIA_Y)
    return pl.pallas_call(
        body,
        out_shape=jax.ShapeDtypeStruct((m, n_out), x.dtype),
        in_specs=[pl.BlockSpec(memory_space=pltpu.VMEM)],
        out_specs=pl.BlockSpec(memory_space=pltpu.VMEM),
        scratch_shapes=[
            pltpu.VMEM((quarter, n_out), x.dtype),
            pltpu.VMEM((n_y * chunk, n_out), x.dtype),
            pltpu.VMEM((quarter, n_out), x.dtype),
            pltpu.VMEM((quarter, n_out), x.dtype),
            pltpu.VMEM((n_x * chunk, n_out), x.dtype),
            pltpu.VMEM((n_z * chunk, n_out), x.dtype),
            pltpu.SemaphoreType.DMA((N_CHUNKS,)),
            pltpu.SemaphoreType.DMA((N_CHUNKS,)),
            pltpu.SemaphoreType.DMA((n_y,)),
            pltpu.SemaphoreType.DMA((n_y,)),
            pltpu.SemaphoreType.DMA((N_CHUNKS,)),
            pltpu.SemaphoreType.DMA((N_CHUNKS,)),
            pltpu.SemaphoreType.DMA((N_CHUNKS,)),
            pltpu.SemaphoreType.DMA((N_CHUNKS,)),
            pltpu.SemaphoreType.DMA((n_x,)),
            pltpu.SemaphoreType.DMA((n_x,)),
            pltpu.SemaphoreType.DMA((n_z,)),
            pltpu.SemaphoreType.DMA((n_z,)),
        ],
        compiler_params=pltpu.CompilerParams(collective_id=0),
    )(x)
